# Initial kernel scaffold; baseline (speedup 1.0000x reference)
import jax
import jax.numpy as jnp
from jax.experimental import pallas as pl


def kernel(x, Wg, bg, W1, b1, W2, b2):
    raise NotImplementedError("write your pallas kernel here")


# speedup vs baseline: 1.8954x; 1.8954x over previous
import functools

import jax
import jax.numpy as jnp
from jax import lax
from jax.experimental import pallas as pl
from jax.experimental.pallas import tpu as pltpu

B = 4096
D = 1024
H = 2048
C = 1024
E = 8

GATE_BLK = 128
NGB = B // GATE_BLK

DENSE_BLK = 256
NDB = B // DENSE_BLK

_NEG_INF = float("-inf")
_INV_SQRT2 = 0.7071067811865476


def _gate_body(x_ref, wgt_ref, bg_ref, gw_ref, i1_ref, i2_ref, w1_ref, w2_ref,
               cnt_ref):
    xb = x_ref[...]
    logits = jnp.dot(xb, wgt_ref[...], preferred_element_type=jnp.float32)
    logits = logits + bg_ref[0:1, :]
    col = lax.broadcasted_iota(jnp.int32, logits.shape, 1)
    l0 = jnp.where(col < E, logits, _NEG_INF)
    v1 = jnp.max(l0, axis=1, keepdims=True)
    i1 = jnp.min(jnp.where(l0 == v1, col, 2**30), axis=1, keepdims=True)
    l1 = jnp.where(col == i1, _NEG_INF, l0)
    v2 = jnp.max(l1, axis=1, keepdims=True)
    i2 = jnp.min(jnp.where(l1 == v2, col, 2**30), axis=1, keepdims=True)
    t = jnp.exp(v2 - v1)
    w1 = 1.0 / (1.0 + t)
    w2 = t / (1.0 + t)
    cols8 = lax.broadcasted_iota(jnp.int32, (GATE_BLK, E), 1)
    gw_ref[...] = (jnp.where(cols8 == i1, w1, 0.0)
                   + jnp.where(cols8 == i2, w2, 0.0))
    i1_ref[...] = i1
    i2_ref[...] = i2
    w1_ref[...] = w1
    w2_ref[...] = w2
    hit = jnp.logical_or(col == i1, col == i2).astype(jnp.int32)
    cnt_ref[...] = jnp.sum(hit, axis=0, keepdims=True).reshape(1, 1, 128)


def _gate(x, wgt_pad, bg_pad):
    return pl.pallas_call(
        _gate_body,
        grid=(NGB,),
        in_specs=[
            pl.BlockSpec((GATE_BLK, D), lambda i: (i, 0)),
            pl.BlockSpec((D, 128), lambda i: (0, 0)),
            pl.BlockSpec((8, 128), lambda i: (0, 0)),
        ],
        out_specs=[
            pl.BlockSpec((GATE_BLK, E), lambda i: (i, 0)),
            pl.BlockSpec((GATE_BLK, 1), lambda i: (i, 0)),
            pl.BlockSpec((GATE_BLK, 1), lambda i: (i, 0)),
            pl.BlockSpec((GATE_BLK, 1), lambda i: (i, 0)),
            pl.BlockSpec((GATE_BLK, 1), lambda i: (i, 0)),
            pl.BlockSpec((1, 1, 128), lambda i: (i, 0, 0)),
        ],
        out_shape=[
            jax.ShapeDtypeStruct((B, E), jnp.float32),
            jax.ShapeDtypeStruct((B, 1), jnp.int32),
            jax.ShapeDtypeStruct((B, 1), jnp.int32),
            jax.ShapeDtypeStruct((B, 1), jnp.float32),
            jax.ShapeDtypeStruct((B, 1), jnp.float32),
            jax.ShapeDtypeStruct((NGB, 1, 128), jnp.int32),
        ],
    )(x, wgt_pad, bg_pad)


def _gelu_exact(h):
    return 0.5 * h * (1.0 + lax.erf(h * _INV_SQRT2))


def _dense_body(x_ref, gw_ref, w1_ref, b1_ref, w2_ref, b2_ref, out_ref):
    e = pl.program_id(1)
    xb = x_ref[...]
    w1 = w1_ref[...].reshape(H, D)
    h = lax.dot_general(xb, w1, (((1,), (1,)), ((), ())),
                        preferred_element_type=jnp.float32)
    h = h + b1_ref[...].reshape(1, H)
    h = _gelu_exact(h)
    w2 = w2_ref[...].reshape(C, H)
    y = lax.dot_general(h, w2, (((1,), (1,)), ((), ())),
                        preferred_element_type=jnp.float32)
    y = y + b2_ref[...].reshape(1, C)
    cols8 = lax.broadcasted_iota(jnp.int32, (DENSE_BLK, E), 1)
    ge = jnp.sum(jnp.where(cols8 == e, gw_ref[...], 0.0), axis=1,
                 keepdims=True)
    contrib = ge * y

    @pl.when(e == 0)
    def _():
        out_ref[...] = contrib

    @pl.when(e > 0)
    def _():
        out_ref[...] = out_ref[...] + contrib


def _dense_experts(x, gw, W1, b1, W2, b2):
    return pl.pallas_call(
        _dense_body,
        grid=(NDB, E),
        in_specs=[
            pl.BlockSpec((DENSE_BLK, D), lambda b, e: (b, 0)),
            pl.BlockSpec((DENSE_BLK, E), lambda b, e: (b, 0)),
            pl.BlockSpec((1, H, D), lambda b, e: (e, 0, 0)),
            pl.BlockSpec((1, 1, H), lambda b, e: (e, 0, 0)),
            pl.BlockSpec((1, C, H), lambda b, e: (e, 0, 0)),
            pl.BlockSpec((1, 1, C), lambda b, e: (e, 0, 0)),
        ],
        out_specs=pl.BlockSpec((DENSE_BLK, C), lambda b, e: (b, 0)),
        out_shape=jax.ShapeDtypeStruct((B, C), jnp.float32),
        compiler_params=pltpu.CompilerParams(
            dimension_semantics=("parallel", "arbitrary"),
        ),
    )(x, gw, W1, b1.reshape(E, 1, H), W2, b2.reshape(E, 1, C))


def kernel(x, Wg, bg, W1, b1, W2, b2):
    wgt_pad = jnp.zeros((D, 128), jnp.float32).at[:, :E].set(Wg.T)
    bg_pad = jnp.zeros((8, 128), jnp.float32).at[0, :E].set(bg)
    gw, i1, i2, w1c, w2c, counts = _gate(x, wgt_pad, bg_pad)
    out = _dense_experts(x, gw, W1, b1, W2, b2)
    return (out, gw)
